# baseline (speedup 1.0000x reference)
import functools

import jax
import jax.numpy as jnp
from jax import lax
from jax.experimental import pallas as pl
from jax.experimental.pallas import tpu as pltpu
from jax.experimental.pallas import tpu_sc as plsc

N = 10000
E = 320000
K = 3
F_IN = 128
H = 512
C = 40

EB = 128
R = 2560
NREG = 320

_MESH = dict(core_axis_name="c", subcore_axis_name="s",
             num_cores=2, num_subcores=16)


def _make_hop(nc):
    mesh = plsc.VectorSubcoreMesh(**_MESH)

    @functools.partial(
        pl.kernel,
        out_type=jax.ShapeDtypeStruct((nc * N, 128), jnp.float32),
        mesh=mesh,
        compiler_params=pltpu.CompilerParams(needs_layout_passes=False),
        scratch_types=dict(
            acc=pltpu.VMEM((NREG, 128), jnp.float32),
            rowbuf=pltpu.VMEM((4, EB, 128), jnp.float32),
            sbs=pltpu.VMEM((16, EB), jnp.int32),
            sbd=pltpu.VMEM((16, EB), jnp.int32),
            sbn=pltpu.VMEM((16, EB), jnp.float32),
            bnd=pltpu.VMEM((8, 128), jnp.int32),
            gsem=pltpu.SemaphoreType.DMA((4,)),
            stsem=pltpu.SemaphoreType.DMA,
        ),
    )
    def hop(h_hbm, srcs_hbm, dst_hbm, nrm_hbm, bounds_hbm, out_hbm,
            acc, rowbuf, sbs, sbd, sbn, bnd, gsem, stsem):
        c = lax.axis_index("c")
        tid = lax.axis_index("s")
        w = c * 16 + tid
        nbase = w * NREG
        pltpu.sync_copy(bounds_hbm.at[pl.ds(pl.multiple_of(w * 8, 8), 8)], bnd)
        bv = bnd[0, pl.ds(0, 16)]
        lo = bv[0]
        hi = bv[1]
        rlo8 = (lo // 1024) * 8
        rhi = (hi + EB - 1) // EB
        nr = jnp.maximum(rhi - rlo8, 0)
        nrr = ((nr + 3) // 4) * 4
        iota16 = lax.iota(jnp.int32, 16)
        zeros = jnp.zeros((16,), jnp.float32)

        def stage(row0, n8off):
            r0 = pl.multiple_of(rlo8 + row0, 8)
            rc = pl.multiple_of(_ch[0] * R + rlo8 + row0, 8)
            no = pl.multiple_of(n8off, 8)
            return (
                pltpu.make_async_copy(
                    srcs_hbm.at[pl.ds(rc, 8)], sbs.at[pl.ds(no, 8)], stsem),
                pltpu.make_async_copy(
                    dst_hbm.at[pl.ds(r0, 8)], sbd.at[pl.ds(no, 8)], stsem),
                pltpu.make_async_copy(
                    nrm_hbm.at[pl.ds(r0, 8)], sbn.at[pl.ds(no, 8)], stsem),
            )

        _ch = [None]

        @pl.loop(0, nc)
        def _chunks(ch):
            _ch[0] = ch

            @pl.loop(0, NREG)
            def _zero(i):
                for k in range(8):
                    acc[i, pl.ds(k * 16, 16)] = zeros

            for d in stage(0, 0):
                d.start()
                d.wait()

            @pl.when(8 < nr)
            def _():
                for d in stage(8, 8):
                    d.start()

            for s0 in range(4):
                @pl.when(s0 < nr)
                def _():
                    pltpu.async_copy(h_hbm.at[sbs.at[s0]],
                                     rowbuf.at[s0], gsem.at[s0])

            @pl.loop(0, nrr, step=4)
            def _slots(S):
                m8 = lax.rem(S, 8)

                @pl.when(jnp.logical_and(m8 == 0,
                                         jnp.logical_and(S > 0, S + 8 < nr)))
                def _():
                    for d in stage(S + 8, lax.rem(S + 8, 16)):
                        d.start()

                @pl.when(jnp.logical_and(m8 == 4, S + 4 < nr))
                def _():
                    for d in stage(S + 4, lax.rem(S + 4, 16)):
                        d.wait()

                for b in range(4):
                    s = S + b

                    @pl.when(s < nr)
                    def _():
                        r16 = lax.rem(s, 16)
                        pltpu.make_async_copy(h_hbm.at[sbs.at[r16]],
                                              rowbuf.at[b], gsem.at[b]).wait()
                        ebase = (rlo8 + s) * EB

                        @pl.loop(0, 8)
                        def _groups(g):
                            sl = pl.ds(g * 16, 16)
                            nv = sbn[r16, sl]
                            d16 = sbd[r16, sl]
                            eidx = ebase + g * 16 + iota16
                            in16 = jnp.logical_and(eidx >= lo, eidx < hi)
                            nvm = jnp.where(in16, nv, 0.0)
                            dv = jnp.clip(d16 - nbase, 0, NREG - 1)
                            for j in range(16):
                                ns = jnp.take(nvm, jnp.full((16,), j,
                                                            jnp.int32))
                                dl = dv[j]
                                e = g * 16 + j
                                for k in range(8):
                                    ksl = pl.ds(k * 16, 16)
                                    plsc.addupdate(
                                        acc.at[dl, ksl],
                                        rowbuf[b, e, ksl] * ns)

                        @pl.when(s + 4 < nr)
                        def _():
                            r16n = lax.rem(s + 4, 16)
                            pltpu.async_copy(h_hbm.at[sbs.at[r16n]],
                                             rowbuf.at[b], gsem.at[b])

            @pl.when(w < 31)
            def _():
                pltpu.sync_copy(
                    acc.at[pl.ds(0, NREG)],
                    out_hbm.at[pl.ds(pl.multiple_of(ch * N + nbase, 8),
                                     NREG)])

            @pl.when(w == 31)
            def _():
                pltpu.sync_copy(
                    acc.at[pl.ds(0, 80)],
                    out_hbm.at[pl.ds(pl.multiple_of(ch * N + 9920, 8), 80)])

    return hop


def _make_norm():
    mesh = plsc.VectorSubcoreMesh(**_MESH)
    RT32 = R // 32

    @functools.partial(
        pl.kernel,
        out_type=jax.ShapeDtypeStruct((R, EB), jnp.float32),
        mesh=mesh,
        compiler_params=pltpu.CompilerParams(needs_layout_passes=False),
        scratch_types=dict(
            disv=pltpu.VMEM((N,), jnp.float32),
            srcv=pltpu.VMEM((RT32, EB), jnp.int32),
            dstv=pltpu.VMEM((RT32, EB), jnp.int32),
            eav=pltpu.VMEM((RT32, EB), jnp.float32),
            outv=pltpu.VMEM((RT32, EB), jnp.float32),
        ),
    )
    def norm_k(dis_hbm, src_hbm, dst_hbm, ea_hbm, nrm_hbm,
               disv, srcv, dstv, eav, outv):
        c = lax.axis_index("c")
        tid = lax.axis_index("s")
        gtid = c * 16 + tid
        rb = gtid * RT32
        pltpu.sync_copy(dis_hbm, disv)
        pltpu.sync_copy(src_hbm.at[pl.ds(rb, RT32)], srcv)
        pltpu.sync_copy(dst_hbm.at[pl.ds(rb, RT32)], dstv)
        pltpu.sync_copy(ea_hbm.at[pl.ds(rb, RT32)], eav)

        @pl.loop(0, RT32)
        def _rows(r):
            for g in range(EB // 16):
                sl = pl.ds(g * 16, 16)
                s16 = srcv[r, sl]
                d16 = dstv[r, sl]
                a16 = eav[r, sl]
                nv = (plsc.load_gather(disv, [s16]) * a16
                      * plsc.load_gather(disv, [d16]))
                outv[r, sl] = nv

        pltpu.sync_copy(outv, nrm_hbm.at[pl.ds(rb, RT32)])

    return norm_k


def _dis_tc(deg):
    def body(d_ref, o_ref):
        d = d_ref[...]
        o_ref[...] = jnp.where(d > 0, lax.rsqrt(jnp.maximum(d, 1e-12)), 0.0)

    return pl.pallas_call(
        body,
        out_shape=jax.ShapeDtypeStruct((N, 128), jnp.float32),
        grid=(5,),
        in_specs=[pl.BlockSpec((2000, 128), lambda i: (i, 0))],
        out_specs=pl.BlockSpec((2000, 128), lambda i: (i, 0)),
    )(deg)


def _layer_tc(hs, W, b, nc_in, nc_out, apply_elu):
    Ho = 128 * nc_out
    BR = 1000

    def body(h0, h1, h2, h3, w_ref, b_ref, o_ref):
        hrefs = (h0, h1, h2, h3)
        acc = jnp.zeros((BR, Ho), jnp.float32)
        for t in range(4):
            for cc in range(nc_in):
                acc = acc + lax.dot_general(
                    hrefs[t][cc], w_ref[t, cc],
                    (((1,), (0,)), ((), ())),
                    precision=lax.Precision.HIGHEST,
                    preferred_element_type=jnp.float32)
        z = acc + b_ref[0][None, :]
        if apply_elu:
            z = jnp.where(z > 0, z, jnp.exp(jnp.minimum(z, 0.0)) - 1.0)
        for co in range(nc_out):
            o_ref[co] = z[:, co * 128:(co + 1) * 128]

    hspec = pl.BlockSpec((nc_in, BR, 128), lambda i: (0, i, 0))
    return pl.pallas_call(
        body,
        out_shape=jax.ShapeDtypeStruct((nc_out, N, 128), jnp.float32),
        grid=(N // BR,),
        in_specs=[hspec, hspec, hspec, hspec,
                  pl.BlockSpec((4, nc_in, 128, Ho), lambda i: (0, 0, 0, 0)),
                  pl.BlockSpec((1, Ho), lambda i: (0, 0))],
        out_specs=pl.BlockSpec((nc_out, BR, 128), lambda i: (0, i, 0)),
    )(*hs, W, b)


def kernel(x, edge_index, edge_attr, W1, b1, Wm, bm, W9, b9):
    perm = jnp.argsort(edge_index[1])
    src = edge_index[0][perm]
    dst = edge_index[1][perm]
    ea = edge_attr[perm]
    pad = R * EB - E
    src_p = jnp.concatenate([src, jnp.zeros((pad,), jnp.int32)])
    dst_p = jnp.concatenate([dst, jnp.zeros((pad,), jnp.int32)]).reshape(R, EB)
    ea_p = jnp.concatenate([ea, jnp.zeros((pad,), jnp.float32)]).reshape(R, EB)
    src1 = src_p.reshape(R, EB)
    src4 = (src_p[None, :]
            + (jnp.arange(4, dtype=jnp.int32) * N)[:, None]).reshape(4 * R, EB)
    svals = jnp.searchsorted(
        dst, jnp.arange(33, dtype=jnp.int32) * NREG).astype(jnp.int32)
    bounds = jnp.zeros((256, 128), jnp.int32)
    rows8 = jnp.arange(32) * 8
    bounds = (bounds.at[rows8, 0].set(svals[:32])
              .at[rows8, 1].set(svals[1:]))

    hop1 = _make_hop(1)
    hop4 = _make_hop(4)
    norm_k = _make_norm()

    ones = jnp.ones((N, 128), jnp.float32)
    deg = hop1(ones, src1, dst_p, ea_p, bounds)
    dis = _dis_tc(deg)
    dis_col = lax.slice(dis, (0, 0), (N, 1)).reshape(N)
    nrm = norm_k(dis_col, src1, dst_p, ea_p)

    h0 = x
    h1 = hop1(h0, src1, dst_p, nrm, bounds)
    h2 = hop1(h1, src1, dst_p, nrm, bounds)
    h3 = hop1(h2, src1, dst_p, nrm, bounds)
    W1r = W1.reshape(4, 1, 128, H)
    h = _layer_tc([h0.reshape(1, N, 128), h1.reshape(1, N, 128),
                   h2.reshape(1, N, 128), h3.reshape(1, N, 128)],
                  W1r, b1.reshape(1, H), 1, 4, True)

    for i in range(7):
        f0 = h.reshape(4 * N, 128)
        f1 = hop4(f0, src4, dst_p, nrm, bounds)
        f2 = hop4(f1, src4, dst_p, nrm, bounds)
        f3 = hop4(f2, src4, dst_p, nrm, bounds)
        Wr = Wm[i].reshape(4, 4, 128, H)
        h = _layer_tc([h, f1.reshape(4, N, 128), f2.reshape(4, N, 128),
                       f3.reshape(4, N, 128)],
                      Wr, bm[i].reshape(1, H), 4, 4, True)

    f0 = h.reshape(4 * N, 128)
    f1 = hop4(f0, src4, dst_p, nrm, bounds)
    f2 = hop4(f1, src4, dst_p, nrm, bounds)
    f3 = hop4(f2, src4, dst_p, nrm, bounds)
    W9p = jnp.pad(W9, ((0, 0), (0, 0), (0, 128 - C)))
    b9p = jnp.pad(b9, (0, 128 - C))
    out = _layer_tc([h, f1.reshape(4, N, 128), f2.reshape(4, N, 128),
                     f3.reshape(4, N, 128)],
                    W9p.reshape(4, 4, 128, 128), b9p.reshape(1, 128),
                    4, 1, False)
    return out.reshape(N, 128)[:, :C]


# speedup vs baseline: 1.0002x; 1.0002x over previous
import functools

import jax
import jax.numpy as jnp
from jax import lax
from jax.experimental import pallas as pl
from jax.experimental.pallas import tpu as pltpu
from jax.experimental.pallas import tpu_sc as plsc

N = 10000
E = 320000
K = 3
F_IN = 128
H = 512
C = 40

EB = 128
R = 2560
NREG = 320

_MESH = dict(core_axis_name="c", subcore_axis_name="s",
             num_cores=2, num_subcores=16)


def _make_hop(nc):
    mesh = plsc.VectorSubcoreMesh(**_MESH)

    @functools.partial(
        pl.kernel,
        out_type=jax.ShapeDtypeStruct((nc * N, 128), jnp.float32),
        mesh=mesh,
        compiler_params=pltpu.CompilerParams(needs_layout_passes=False),
        scratch_types=dict(
            acc=pltpu.VMEM((NREG, 128), jnp.float32),
            rowbuf=pltpu.VMEM((4, EB, 128), jnp.float32),
            sbs=pltpu.VMEM((16, EB), jnp.int32),
            sbd=pltpu.VMEM((16, EB), jnp.int32),
            sbn=pltpu.VMEM((16, EB), jnp.float32),
            bnd=pltpu.VMEM((8, 128), jnp.int32),
            gsem=pltpu.SemaphoreType.DMA((4,)),
            stsem=pltpu.SemaphoreType.DMA,
        ),
    )
    def hop(h_hbm, srcs_hbm, dst_hbm, nrm_hbm, bounds_hbm, out_hbm,
            acc, rowbuf, sbs, sbd, sbn, bnd, gsem, stsem):
        c = lax.axis_index("c")
        tid = lax.axis_index("s")
        w = c * 16 + tid
        nbase = w * NREG
        pltpu.sync_copy(bounds_hbm.at[pl.ds(pl.multiple_of(w * 8, 8), 8)], bnd)
        bv = bnd[0, pl.ds(0, 16)]
        lo = bv[0]
        hi = bv[1]
        rlo8 = (lo // 1024) * 8
        rhi = (hi + EB - 1) // EB
        nr = jnp.maximum(rhi - rlo8, 0)
        nrr = ((nr + 3) // 4) * 4
        iota16 = lax.iota(jnp.int32, 16)
        zeros = jnp.zeros((16,), jnp.float32)

        def stage(row0, n8off):
            r0 = pl.multiple_of(rlo8 + row0, 8)
            rc = pl.multiple_of(_ch[0] * R + rlo8 + row0, 8)
            no = pl.multiple_of(n8off, 8)
            return (
                pltpu.make_async_copy(
                    srcs_hbm.at[pl.ds(rc, 8)], sbs.at[pl.ds(no, 8)], stsem),
                pltpu.make_async_copy(
                    dst_hbm.at[pl.ds(r0, 8)], sbd.at[pl.ds(no, 8)], stsem),
                pltpu.make_async_copy(
                    nrm_hbm.at[pl.ds(r0, 8)], sbn.at[pl.ds(no, 8)], stsem),
            )

        _ch = [None]

        @pl.loop(0, nc)
        def _chunks(ch):
            _ch[0] = ch

            @pl.loop(0, NREG)
            def _zero(i):
                for k in range(8):
                    acc[i, pl.ds(k * 16, 16)] = zeros

            for d in stage(0, 0):
                d.start()
                d.wait()

            @pl.when(8 < nr)
            def _():
                for d in stage(8, 8):
                    d.start()

            for s0 in range(4):
                @pl.when(s0 < nr)
                def _():
                    pltpu.async_copy(h_hbm.at[sbs.at[s0]],
                                     rowbuf.at[s0], gsem.at[s0])

            @pl.loop(0, nrr, step=4)
            def _slots(S):
                m8 = lax.rem(S, 8)

                @pl.when(jnp.logical_and(m8 == 0,
                                         jnp.logical_and(S > 0, S + 8 < nr)))
                def _():
                    for d in stage(S + 8, lax.rem(S + 8, 16)):
                        d.start()

                @pl.when(jnp.logical_and(m8 == 4, S + 4 < nr))
                def _():
                    for d in stage(S + 4, lax.rem(S + 4, 16)):
                        d.wait()

                for b in range(4):
                    s = S + b

                    @pl.when(s < nr)
                    def _():
                        r16 = lax.rem(s, 16)
                        pltpu.make_async_copy(h_hbm.at[sbs.at[r16]],
                                              rowbuf.at[b], gsem.at[b]).wait()
                        ebase = (rlo8 + s) * EB

                        @pl.loop(0, 8)
                        def _groups(g):
                            sl = pl.ds(g * 16, 16)
                            nv = sbn[r16, sl]
                            d16 = sbd[r16, sl]
                            eidx = ebase + g * 16 + iota16
                            in16 = jnp.logical_and(eidx >= lo, eidx < hi)
                            nvm = jnp.where(in16, nv, 0.0)
                            dv = jnp.clip(d16 - nbase, 0, NREG - 1)
                            for j in range(16):
                                ns = jnp.take(nvm, jnp.full((16,), j,
                                                            jnp.int32))
                                dl = dv[j]
                                e = g * 16 + j
                                for k in range(8):
                                    ksl = pl.ds(k * 16, 16)
                                    plsc.addupdate(
                                        acc.at[dl, ksl],
                                        rowbuf[b, e, ksl] * ns)

                        @pl.when(s + 4 < nr)
                        def _():
                            r16n = lax.rem(s + 4, 16)
                            pltpu.async_copy(h_hbm.at[sbs.at[r16n]],
                                             rowbuf.at[b], gsem.at[b])

            @pl.when(w < 31)
            def _():
                pltpu.sync_copy(
                    acc.at[pl.ds(0, NREG)],
                    out_hbm.at[pl.ds(pl.multiple_of(ch * N + nbase, 8),
                                     NREG)])

            @pl.when(w == 31)
            def _():
                pltpu.sync_copy(
                    acc.at[pl.ds(0, 80)],
                    out_hbm.at[pl.ds(pl.multiple_of(ch * N + 9920, 8), 80)])

    return hop


def _make_norm():
    mesh = plsc.VectorSubcoreMesh(**_MESH)
    RT32 = R // 32

    @functools.partial(
        pl.kernel,
        out_type=jax.ShapeDtypeStruct((R, EB), jnp.float32),
        mesh=mesh,
        compiler_params=pltpu.CompilerParams(needs_layout_passes=False),
        scratch_types=dict(
            disv=pltpu.VMEM((N,), jnp.float32),
            srcv=pltpu.VMEM((RT32, EB), jnp.int32),
            dstv=pltpu.VMEM((RT32, EB), jnp.int32),
            eav=pltpu.VMEM((RT32, EB), jnp.float32),
            outv=pltpu.VMEM((RT32, EB), jnp.float32),
        ),
    )
    def norm_k(dis_hbm, src_hbm, dst_hbm, ea_hbm, nrm_hbm,
               disv, srcv, dstv, eav, outv):
        c = lax.axis_index("c")
        tid = lax.axis_index("s")
        gtid = c * 16 + tid
        rb = gtid * RT32
        pltpu.sync_copy(dis_hbm, disv)
        pltpu.sync_copy(src_hbm.at[pl.ds(rb, RT32)], srcv)
        pltpu.sync_copy(dst_hbm.at[pl.ds(rb, RT32)], dstv)
        pltpu.sync_copy(ea_hbm.at[pl.ds(rb, RT32)], eav)

        @pl.loop(0, RT32)
        def _rows(r):
            for g in range(EB // 16):
                sl = pl.ds(g * 16, 16)
                s16 = srcv[r, sl]
                d16 = dstv[r, sl]
                a16 = eav[r, sl]
                nv = (plsc.load_gather(disv, [s16]) * a16
                      * plsc.load_gather(disv, [d16]))
                outv[r, sl] = nv

        pltpu.sync_copy(outv, nrm_hbm.at[pl.ds(rb, RT32)])

    return norm_k


def _dis_tc(deg):
    def body(d_ref, o_ref):
        d = d_ref[...]
        o_ref[...] = jnp.where(d > 0, lax.rsqrt(jnp.maximum(d, 1e-12)), 0.0)

    return pl.pallas_call(
        body,
        out_shape=jax.ShapeDtypeStruct((N, 128), jnp.float32),
        grid=(5,),
        in_specs=[pl.BlockSpec((2000, 128), lambda i: (i, 0))],
        out_specs=pl.BlockSpec((2000, 128), lambda i: (i, 0)),
    )(deg)


def _layer_tc(hs, W, b, nc_in, nc_out, apply_elu):
    Ho = 128 * nc_out
    BR = 1000

    def body(h0, h1, h2, h3, w_ref, b_ref, o_ref):
        hrefs = (h0, h1, h2, h3)
        acc = jnp.zeros((BR, Ho), jnp.float32)
        for t in range(4):
            for cc in range(nc_in):
                acc = acc + lax.dot_general(
                    hrefs[t][cc], w_ref[t, cc],
                    (((1,), (0,)), ((), ())),
                    precision=lax.Precision.HIGHEST,
                    preferred_element_type=jnp.float32)
        z = acc + b_ref[0][None, :]
        if apply_elu:
            z = jnp.where(z > 0, z, jnp.exp(jnp.minimum(z, 0.0)) - 1.0)
        for co in range(nc_out):
            o_ref[co] = z[:, co * 128:(co + 1) * 128]

    hspec = pl.BlockSpec((nc_in, BR, 128), lambda i: (0, i, 0))
    return pl.pallas_call(
        body,
        out_shape=jax.ShapeDtypeStruct((nc_out, N, 128), jnp.float32),
        grid=(N // BR,),
        in_specs=[hspec, hspec, hspec, hspec,
                  pl.BlockSpec((4, nc_in, 128, Ho), lambda i: (0, 0, 0, 0)),
                  pl.BlockSpec((1, Ho), lambda i: (0, 0))],
        out_specs=pl.BlockSpec((nc_out, BR, 128), lambda i: (0, i, 0)),
    )(*hs, W, b)


def kernel(x, edge_index, edge_attr, W1, b1, Wm, bm, W9, b9):
    key = (edge_index[1] // NREG) * 16384 + edge_index[0]
    perm = jnp.argsort(key)
    src = edge_index[0][perm]
    dst = edge_index[1][perm]
    ea = edge_attr[perm]
    pad = R * EB - E
    src_p = jnp.concatenate([src, jnp.zeros((pad,), jnp.int32)])
    dst_p = jnp.concatenate([dst, jnp.zeros((pad,), jnp.int32)]).reshape(R, EB)
    ea_p = jnp.concatenate([ea, jnp.zeros((pad,), jnp.float32)]).reshape(R, EB)
    src1 = src_p.reshape(R, EB)
    src4 = (src_p[None, :]
            + (jnp.arange(4, dtype=jnp.int32) * N)[:, None]).reshape(4 * R, EB)
    svals = jnp.searchsorted(
        dst // NREG, jnp.arange(33, dtype=jnp.int32)).astype(jnp.int32)
    bounds = jnp.zeros((256, 128), jnp.int32)
    rows8 = jnp.arange(32) * 8
    bounds = (bounds.at[rows8, 0].set(svals[:32])
              .at[rows8, 1].set(svals[1:]))

    hop1 = _make_hop(1)
    hop4 = _make_hop(4)
    norm_k = _make_norm()

    ones = jnp.ones((N, 128), jnp.float32)
    deg = hop1(ones, src1, dst_p, ea_p, bounds)
    dis = _dis_tc(deg)
    dis_col = lax.slice(dis, (0, 0), (N, 1)).reshape(N)
    nrm = norm_k(dis_col, src1, dst_p, ea_p)

    h0 = x
    h1 = hop1(h0, src1, dst_p, nrm, bounds)
    h2 = hop1(h1, src1, dst_p, nrm, bounds)
    h3 = hop1(h2, src1, dst_p, nrm, bounds)
    W1r = W1.reshape(4, 1, 128, H)
    h = _layer_tc([h0.reshape(1, N, 128), h1.reshape(1, N, 128),
                   h2.reshape(1, N, 128), h3.reshape(1, N, 128)],
                  W1r, b1.reshape(1, H), 1, 4, True)

    for i in range(7):
        f0 = h.reshape(4 * N, 128)
        f1 = hop4(f0, src4, dst_p, nrm, bounds)
        f2 = hop4(f1, src4, dst_p, nrm, bounds)
        f3 = hop4(f2, src4, dst_p, nrm, bounds)
        Wr = Wm[i].reshape(4, 4, 128, H)
        h = _layer_tc([h, f1.reshape(4, N, 128), f2.reshape(4, N, 128),
                       f3.reshape(4, N, 128)],
                      Wr, bm[i].reshape(1, H), 4, 4, True)

    f0 = h.reshape(4 * N, 128)
    f1 = hop4(f0, src4, dst_p, nrm, bounds)
    f2 = hop4(f1, src4, dst_p, nrm, bounds)
    f3 = hop4(f2, src4, dst_p, nrm, bounds)
    W9p = jnp.pad(W9, ((0, 0), (0, 0), (0, 128 - C)))
    b9p = jnp.pad(b9, (0, 128 - C))
    out = _layer_tc([h, f1.reshape(4, N, 128), f2.reshape(4, N, 128),
                     f3.reshape(4, N, 128)],
                    W9p.reshape(4, 4, 128, 128), b9p.reshape(1, 128),
                    4, 1, False)
    return out.reshape(N, 128)[:, :C]


# speedup vs baseline: 3.1301x; 3.1294x over previous
import functools

import jax
import jax.numpy as jnp
from jax import lax
from jax.experimental import pallas as pl
from jax.experimental.pallas import tpu as pltpu
from jax.experimental.pallas import tpu_sc as plsc

N = 10000
E = 320000
K = 3
F_IN = 128
H = 512
C = 40

EB = 128
R = 2560
NREG = 320

_MESH = dict(core_axis_name="c", subcore_axis_name="s",
             num_cores=2, num_subcores=16)


def _make_hop(nc):
    mesh = plsc.VectorSubcoreMesh(**_MESH)

    @functools.partial(
        pl.kernel,
        out_type=jax.ShapeDtypeStruct((nc * N, 128), jnp.float32),
        mesh=mesh,
        compiler_params=pltpu.CompilerParams(needs_layout_passes=False),
        scratch_types=dict(
            acc=pltpu.VMEM((NREG, 128), jnp.float32),
            rowbuf=pltpu.VMEM((4, EB, 128), jnp.float32),
            sbs=pltpu.VMEM((16, EB), jnp.int32),
            sbd=pltpu.VMEM((16, EB), jnp.int32),
            sbn=pltpu.VMEM((16, EB), jnp.float32),
            bnd=pltpu.VMEM((8, 128), jnp.int32),
            gsem=pltpu.SemaphoreType.DMA((4,)),
            stsem=pltpu.SemaphoreType.DMA,
        ),
    )
    def hop(h_hbm, srcs_hbm, dst_hbm, nrm_hbm, bounds_hbm, out_hbm,
            acc, rowbuf, sbs, sbd, sbn, bnd, gsem, stsem):
        c = lax.axis_index("c")
        tid = lax.axis_index("s")
        w = c * 16 + tid
        nbase = w * NREG
        pltpu.sync_copy(bounds_hbm.at[pl.ds(pl.multiple_of(w * 8, 8), 8)], bnd)
        bv = bnd[0, pl.ds(0, 16)]
        lo = bv[0]
        hi = bv[1]
        rlo8 = (lo // 1024) * 8
        rhi = (hi + EB - 1) // EB
        nr = jnp.maximum(rhi - rlo8, 0)
        nrr = ((nr + 3) // 4) * 4
        iota16 = lax.iota(jnp.int32, 16)
        zeros = jnp.zeros((16,), jnp.float32)

        def stage(row0, n8off):
            r0 = pl.multiple_of(rlo8 + row0, 8)
            rc = pl.multiple_of(_ch[0] * R + rlo8 + row0, 8)
            no = pl.multiple_of(n8off, 8)
            return (
                pltpu.make_async_copy(
                    srcs_hbm.at[pl.ds(rc, 8)], sbs.at[pl.ds(no, 8)], stsem),
                pltpu.make_async_copy(
                    dst_hbm.at[pl.ds(r0, 8)], sbd.at[pl.ds(no, 8)], stsem),
                pltpu.make_async_copy(
                    nrm_hbm.at[pl.ds(r0, 8)], sbn.at[pl.ds(no, 8)], stsem),
            )

        _ch = [None]

        @pl.loop(0, nc)
        def _chunks(ch):
            _ch[0] = ch

            @pl.loop(0, NREG)
            def _zero(i):
                for k in range(8):
                    acc[i, pl.ds(k * 16, 16)] = zeros

            for d in stage(0, 0):
                d.start()
                d.wait()

            @pl.when(8 < nr)
            def _():
                for d in stage(8, 8):
                    d.start()

            for s0 in range(4):
                @pl.when(s0 < nr)
                def _():
                    pltpu.async_copy(h_hbm.at[sbs.at[s0]],
                                     rowbuf.at[s0], gsem.at[s0])

            @pl.loop(0, nrr, step=4)
            def _slots(S):
                m8 = lax.rem(S, 8)

                @pl.when(jnp.logical_and(m8 == 0,
                                         jnp.logical_and(S > 0, S + 8 < nr)))
                def _():
                    for d in stage(S + 8, lax.rem(S + 8, 16)):
                        d.start()

                @pl.when(jnp.logical_and(m8 == 4, S + 4 < nr))
                def _():
                    for d in stage(S + 4, lax.rem(S + 4, 16)):
                        d.wait()

                for b in range(4):
                    s = S + b

                    @pl.when(s < nr)
                    def _():
                        r16 = lax.rem(s, 16)
                        pltpu.make_async_copy(h_hbm.at[sbs.at[r16]],
                                              rowbuf.at[b], gsem.at[b]).wait()
                        ebase = (rlo8 + s) * EB

                        @pl.loop(0, 8)
                        def _groups(g):
                            sl = pl.ds(g * 16, 16)
                            nv = sbn[r16, sl]
                            d16 = sbd[r16, sl]
                            eidx = ebase + g * 16 + iota16
                            in16 = jnp.logical_and(eidx >= lo, eidx < hi)
                            nvm = jnp.where(in16, nv, 0.0)
                            dv = jnp.clip(d16 - nbase, 0, NREG - 1)
                            for j in range(16):
                                ns = jnp.take(nvm, jnp.full((16,), j,
                                                            jnp.int32))
                                dl = dv[j]
                                e = g * 16 + j
                                vals = [rowbuf[b, e, pl.ds(k * 16, 16)] * ns
                                        for k in range(8)]
                                for k in range(8):
                                    plsc.addupdate(
                                        acc.at[dl, pl.ds(k * 16, 16)],
                                        vals[k])

                        @pl.when(s + 4 < nr)
                        def _():
                            r16n = lax.rem(s + 4, 16)
                            pltpu.async_copy(h_hbm.at[sbs.at[r16n]],
                                             rowbuf.at[b], gsem.at[b])

            @pl.when(w < 31)
            def _():
                pltpu.sync_copy(
                    acc.at[pl.ds(0, NREG)],
                    out_hbm.at[pl.ds(pl.multiple_of(ch * N + nbase, 8),
                                     NREG)])

            @pl.when(w == 31)
            def _():
                pltpu.sync_copy(
                    acc.at[pl.ds(0, 80)],
                    out_hbm.at[pl.ds(pl.multiple_of(ch * N + 9920, 8), 80)])

    return hop


def _make_norm():
    mesh = plsc.VectorSubcoreMesh(**_MESH)
    RT32 = R // 32

    @functools.partial(
        pl.kernel,
        out_type=jax.ShapeDtypeStruct((R, EB), jnp.float32),
        mesh=mesh,
        compiler_params=pltpu.CompilerParams(needs_layout_passes=False),
        scratch_types=dict(
            disv=pltpu.VMEM((N,), jnp.float32),
            srcv=pltpu.VMEM((RT32, EB), jnp.int32),
            dstv=pltpu.VMEM((RT32, EB), jnp.int32),
            eav=pltpu.VMEM((RT32, EB), jnp.float32),
            outv=pltpu.VMEM((RT32, EB), jnp.float32),
        ),
    )
    def norm_k(dis_hbm, src_hbm, dst_hbm, ea_hbm, nrm_hbm,
               disv, srcv, dstv, eav, outv):
        c = lax.axis_index("c")
        tid = lax.axis_index("s")
        gtid = c * 16 + tid
        rb = gtid * RT32
        pltpu.sync_copy(dis_hbm, disv)
        pltpu.sync_copy(src_hbm.at[pl.ds(rb, RT32)], srcv)
        pltpu.sync_copy(dst_hbm.at[pl.ds(rb, RT32)], dstv)
        pltpu.sync_copy(ea_hbm.at[pl.ds(rb, RT32)], eav)

        @pl.loop(0, RT32)
        def _rows(r):
            for g in range(EB // 16):
                sl = pl.ds(g * 16, 16)
                s16 = srcv[r, sl]
                d16 = dstv[r, sl]
                a16 = eav[r, sl]
                nv = (plsc.load_gather(disv, [s16]) * a16
                      * plsc.load_gather(disv, [d16]))
                outv[r, sl] = nv

        pltpu.sync_copy(outv, nrm_hbm.at[pl.ds(rb, RT32)])

    return norm_k


def _dis_tc(deg):
    def body(d_ref, o_ref):
        d = d_ref[...]
        o_ref[...] = jnp.where(d > 0, lax.rsqrt(jnp.maximum(d, 1e-12)), 0.0)

    return pl.pallas_call(
        body,
        out_shape=jax.ShapeDtypeStruct((N, 128), jnp.float32),
        grid=(5,),
        in_specs=[pl.BlockSpec((2000, 128), lambda i: (i, 0))],
        out_specs=pl.BlockSpec((2000, 128), lambda i: (i, 0)),
    )(deg)


def _layer_tc(hs, W, b, nc_in, nc_out, apply_elu):
    Ho = 128 * nc_out
    BR = 1000

    def body(h0, h1, h2, h3, w_ref, b_ref, o_ref):
        hrefs = (h0, h1, h2, h3)
        acc = jnp.zeros((BR, Ho), jnp.float32)
        for t in range(4):
            for cc in range(nc_in):
                acc = acc + lax.dot_general(
                    hrefs[t][cc], w_ref[t, cc],
                    (((1,), (0,)), ((), ())),
                    precision=lax.Precision.HIGHEST,
                    preferred_element_type=jnp.float32)
        z = acc + b_ref[0][None, :]
        if apply_elu:
            z = jnp.where(z > 0, z, jnp.exp(jnp.minimum(z, 0.0)) - 1.0)
        for co in range(nc_out):
            o_ref[co] = z[:, co * 128:(co + 1) * 128]

    hspec = pl.BlockSpec((nc_in, BR, 128), lambda i: (0, i, 0))
    return pl.pallas_call(
        body,
        out_shape=jax.ShapeDtypeStruct((nc_out, N, 128), jnp.float32),
        grid=(N // BR,),
        in_specs=[hspec, hspec, hspec, hspec,
                  pl.BlockSpec((4, nc_in, 128, Ho), lambda i: (0, 0, 0, 0)),
                  pl.BlockSpec((1, Ho), lambda i: (0, 0))],
        out_specs=pl.BlockSpec((nc_out, BR, 128), lambda i: (0, i, 0)),
    )(*hs, W, b)


def kernel(x, edge_index, edge_attr, W1, b1, Wm, bm, W9, b9):
    key = (edge_index[1] // NREG) * 16384 + edge_index[0]
    perm = jnp.argsort(key)
    src = edge_index[0][perm]
    dst = edge_index[1][perm]
    ea = edge_attr[perm]
    pad = R * EB - E
    src_p = jnp.concatenate([src, jnp.zeros((pad,), jnp.int32)])
    dst_p = jnp.concatenate([dst, jnp.zeros((pad,), jnp.int32)]).reshape(R, EB)
    ea_p = jnp.concatenate([ea, jnp.zeros((pad,), jnp.float32)]).reshape(R, EB)
    src1 = src_p.reshape(R, EB)
    src4 = (src_p[None, :]
            + (jnp.arange(4, dtype=jnp.int32) * N)[:, None]).reshape(4 * R, EB)
    svals = jnp.searchsorted(
        dst // NREG, jnp.arange(33, dtype=jnp.int32)).astype(jnp.int32)
    bounds = jnp.zeros((256, 128), jnp.int32)
    rows8 = jnp.arange(32) * 8
    bounds = (bounds.at[rows8, 0].set(svals[:32])
              .at[rows8, 1].set(svals[1:]))

    hop1 = _make_hop(1)
    hop4 = _make_hop(4)
    norm_k = _make_norm()

    ones = jnp.ones((N, 128), jnp.float32)
    deg = hop1(ones, src1, dst_p, ea_p, bounds)
    dis = _dis_tc(deg)
    dis_col = lax.slice(dis, (0, 0), (N, 1)).reshape(N)
    nrm = norm_k(dis_col, src1, dst_p, ea_p)

    h0 = x
    h1 = hop1(h0, src1, dst_p, nrm, bounds)
    h2 = hop1(h1, src1, dst_p, nrm, bounds)
    h3 = hop1(h2, src1, dst_p, nrm, bounds)
    W1r = W1.reshape(4, 1, 128, H)
    h = _layer_tc([h0.reshape(1, N, 128), h1.reshape(1, N, 128),
                   h2.reshape(1, N, 128), h3.reshape(1, N, 128)],
                  W1r, b1.reshape(1, H), 1, 4, True)

    for i in range(7):
        f0 = h.reshape(4 * N, 128)
        f1 = hop4(f0, src4, dst_p, nrm, bounds)
        f2 = hop4(f1, src4, dst_p, nrm, bounds)
        f3 = hop4(f2, src4, dst_p, nrm, bounds)
        Wr = Wm[i].reshape(4, 4, 128, H)
        h = _layer_tc([h, f1.reshape(4, N, 128), f2.reshape(4, N, 128),
                       f3.reshape(4, N, 128)],
                      Wr, bm[i].reshape(1, H), 4, 4, True)

    f0 = h.reshape(4 * N, 128)
    f1 = hop4(f0, src4, dst_p, nrm, bounds)
    f2 = hop4(f1, src4, dst_p, nrm, bounds)
    f3 = hop4(f2, src4, dst_p, nrm, bounds)
    W9p = jnp.pad(W9, ((0, 0), (0, 0), (0, 128 - C)))
    b9p = jnp.pad(b9, (0, 128 - C))
    out = _layer_tc([h, f1.reshape(4, N, 128), f2.reshape(4, N, 128),
                     f3.reshape(4, N, 128)],
                    W9p.reshape(4, 4, 128, 128), b9p.reshape(1, 128),
                    4, 1, False)
    return out.reshape(N, 128)[:, :C]


# speedup vs baseline: 3.1951x; 1.0208x over previous
import functools

import jax
import jax.numpy as jnp
from jax import lax
from jax.experimental import pallas as pl
from jax.experimental.pallas import tpu as pltpu
from jax.experimental.pallas import tpu_sc as plsc

N = 10000
E = 320000
K = 3
F_IN = 128
H = 512
C = 40

EB = 128
R = 2560
NREG = 320

_MESH = dict(core_axis_name="c", subcore_axis_name="s",
             num_cores=2, num_subcores=16)


def _make_hop(nc):
    mesh = plsc.VectorSubcoreMesh(**_MESH)

    @functools.partial(
        pl.kernel,
        out_type=jax.ShapeDtypeStruct((nc * N, 128), jnp.float32),
        mesh=mesh,
        compiler_params=pltpu.CompilerParams(needs_layout_passes=False),
        scratch_types=dict(
            acc=pltpu.VMEM((NREG, 128), jnp.float32),
            rowbuf=pltpu.VMEM((4, EB, 128), jnp.float32),
            sbs=pltpu.VMEM((16, EB), jnp.int32),
            sbd=pltpu.VMEM((16, EB), jnp.int32),
            sbn=pltpu.VMEM((16, EB), jnp.float32),
            bnd=pltpu.VMEM((8, 128), jnp.int32),
            gsem=pltpu.SemaphoreType.DMA((4,)),
            stsem=pltpu.SemaphoreType.DMA,
        ),
    )
    def hop(h_hbm, srcs_hbm, dst_hbm, nrm_hbm, bounds_hbm, out_hbm,
            acc, rowbuf, sbs, sbd, sbn, bnd, gsem, stsem):
        c = lax.axis_index("c")
        tid = lax.axis_index("s")
        w = c * 16 + tid
        nbase = w * NREG
        pltpu.sync_copy(bounds_hbm.at[pl.ds(pl.multiple_of(w * 8, 8), 8)], bnd)
        bv = bnd[0, pl.ds(0, 16)]
        lo = bv[0]
        hi = bv[1]
        rlo8 = (lo // 1024) * 8
        rhi = (hi + EB - 1) // EB
        nr = jnp.maximum(rhi - rlo8, 0)
        nrr = ((nr + 3) // 4) * 4
        iota16 = lax.iota(jnp.int32, 16)
        zeros = jnp.zeros((16,), jnp.float32)

        def stage(row0, n8off):
            r0 = pl.multiple_of(rlo8 + row0, 8)
            rc = pl.multiple_of(_ch[0] * R + rlo8 + row0, 8)
            no = pl.multiple_of(n8off, 8)
            return (
                pltpu.make_async_copy(
                    srcs_hbm.at[pl.ds(rc, 8)], sbs.at[pl.ds(no, 8)], stsem),
                pltpu.make_async_copy(
                    dst_hbm.at[pl.ds(r0, 8)], sbd.at[pl.ds(no, 8)], stsem),
                pltpu.make_async_copy(
                    nrm_hbm.at[pl.ds(r0, 8)], sbn.at[pl.ds(no, 8)], stsem),
            )

        _ch = [None]

        @pl.loop(0, nc)
        def _chunks(ch):
            _ch[0] = ch

            @pl.loop(0, NREG)
            def _zero(i):
                for k in range(8):
                    acc[i, pl.ds(k * 16, 16)] = zeros

            for d in stage(0, 0):
                d.start()
                d.wait()

            @pl.when(8 < nr)
            def _():
                for d in stage(8, 8):
                    d.start()

            for s0 in range(4):
                @pl.when(s0 < nr)
                def _():
                    pltpu.async_copy(h_hbm.at[sbs.at[s0]],
                                     rowbuf.at[s0], gsem.at[s0])

            @pl.loop(0, nrr, step=4)
            def _slots(S):
                m8 = lax.rem(S, 8)

                @pl.when(jnp.logical_and(m8 == 0,
                                         jnp.logical_and(S > 0, S + 8 < nr)))
                def _():
                    for d in stage(S + 8, lax.rem(S + 8, 16)):
                        d.start()

                @pl.when(jnp.logical_and(m8 == 4, S + 4 < nr))
                def _():
                    for d in stage(S + 4, lax.rem(S + 4, 16)):
                        d.wait()

                for b in range(4):
                    s = S + b

                    @pl.when(s < nr)
                    def _():
                        r16 = lax.rem(s, 16)
                        pltpu.make_async_copy(h_hbm.at[sbs.at[r16]],
                                              rowbuf.at[b], gsem.at[b]).wait()
                        ebase = (rlo8 + s) * EB

                        @pl.loop(0, 8)
                        def _groups(g):
                            sl = pl.ds(g * 16, 16)
                            nv = sbn[r16, sl]
                            d16 = sbd[r16, sl]
                            eidx = ebase + g * 16 + iota16
                            in16 = jnp.logical_and(eidx >= lo, eidx < hi)
                            nvm = jnp.where(in16, nv, 0.0)
                            dv = jnp.clip(d16 - nbase, 0, NREG - 1)
                            for j in range(0, 16, 2):
                                ns0 = jnp.take(nvm, jnp.full((16,), j,
                                                             jnp.int32))
                                ns1 = jnp.take(nvm, jnp.full((16,), j + 1,
                                                             jnp.int32))
                                dl0 = dv[j]
                                dl1 = dv[j + 1]
                                e0 = g * 16 + j
                                vals = (
                                    [rowbuf[b, e0, pl.ds(k * 16, 16)] * ns0
                                     for k in range(8)]
                                    + [rowbuf[b, e0 + 1, pl.ds(k * 16, 16)]
                                       * ns1 for k in range(8)])
                                for k in range(8):
                                    plsc.addupdate(
                                        acc.at[dl0, pl.ds(k * 16, 16)],
                                        vals[k])
                                for k in range(8):
                                    plsc.addupdate(
                                        acc.at[dl1, pl.ds(k * 16, 16)],
                                        vals[8 + k])

                        @pl.when(s + 4 < nr)
                        def _():
                            r16n = lax.rem(s + 4, 16)
                            pltpu.async_copy(h_hbm.at[sbs.at[r16n]],
                                             rowbuf.at[b], gsem.at[b])

            @pl.when(w < 31)
            def _():
                pltpu.sync_copy(
                    acc.at[pl.ds(0, NREG)],
                    out_hbm.at[pl.ds(pl.multiple_of(ch * N + nbase, 8),
                                     NREG)])

            @pl.when(w == 31)
            def _():
                pltpu.sync_copy(
                    acc.at[pl.ds(0, 80)],
                    out_hbm.at[pl.ds(pl.multiple_of(ch * N + 9920, 8), 80)])

    return hop


def _make_norm():
    mesh = plsc.VectorSubcoreMesh(**_MESH)
    RT32 = R // 32

    @functools.partial(
        pl.kernel,
        out_type=jax.ShapeDtypeStruct((R, EB), jnp.float32),
        mesh=mesh,
        compiler_params=pltpu.CompilerParams(needs_layout_passes=False),
        scratch_types=dict(
            disv=pltpu.VMEM((N,), jnp.float32),
            srcv=pltpu.VMEM((RT32, EB), jnp.int32),
            dstv=pltpu.VMEM((RT32, EB), jnp.int32),
            eav=pltpu.VMEM((RT32, EB), jnp.float32),
            outv=pltpu.VMEM((RT32, EB), jnp.float32),
        ),
    )
    def norm_k(dis_hbm, src_hbm, dst_hbm, ea_hbm, nrm_hbm,
               disv, srcv, dstv, eav, outv):
        c = lax.axis_index("c")
        tid = lax.axis_index("s")
        gtid = c * 16 + tid
        rb = gtid * RT32
        pltpu.sync_copy(dis_hbm, disv)
        pltpu.sync_copy(src_hbm.at[pl.ds(rb, RT32)], srcv)
        pltpu.sync_copy(dst_hbm.at[pl.ds(rb, RT32)], dstv)
        pltpu.sync_copy(ea_hbm.at[pl.ds(rb, RT32)], eav)

        @pl.loop(0, RT32)
        def _rows(r):
            for g in range(EB // 16):
                sl = pl.ds(g * 16, 16)
                s16 = srcv[r, sl]
                d16 = dstv[r, sl]
                a16 = eav[r, sl]
                nv = (plsc.load_gather(disv, [s16]) * a16
                      * plsc.load_gather(disv, [d16]))
                outv[r, sl] = nv

        pltpu.sync_copy(outv, nrm_hbm.at[pl.ds(rb, RT32)])

    return norm_k


def _dis_tc(deg):
    def body(d_ref, o_ref):
        d = d_ref[...]
        o_ref[...] = jnp.where(d > 0, lax.rsqrt(jnp.maximum(d, 1e-12)), 0.0)

    return pl.pallas_call(
        body,
        out_shape=jax.ShapeDtypeStruct((N, 128), jnp.float32),
        grid=(5,),
        in_specs=[pl.BlockSpec((2000, 128), lambda i: (i, 0))],
        out_specs=pl.BlockSpec((2000, 128), lambda i: (i, 0)),
    )(deg)


def _layer_tc(hs, W, b, nc_in, nc_out, apply_elu):
    Ho = 128 * nc_out
    BR = 1000

    def body(h0, h1, h2, h3, w_ref, b_ref, o_ref):
        hrefs = (h0, h1, h2, h3)
        acc = jnp.zeros((BR, Ho), jnp.float32)
        for t in range(4):
            for cc in range(nc_in):
                acc = acc + lax.dot_general(
                    hrefs[t][cc], w_ref[t, cc],
                    (((1,), (0,)), ((), ())),
                    precision=lax.Precision.HIGHEST,
                    preferred_element_type=jnp.float32)
        z = acc + b_ref[0][None, :]
        if apply_elu:
            z = jnp.where(z > 0, z, jnp.exp(jnp.minimum(z, 0.0)) - 1.0)
        for co in range(nc_out):
            o_ref[co] = z[:, co * 128:(co + 1) * 128]

    hspec = pl.BlockSpec((nc_in, BR, 128), lambda i: (0, i, 0))
    return pl.pallas_call(
        body,
        out_shape=jax.ShapeDtypeStruct((nc_out, N, 128), jnp.float32),
        grid=(N // BR,),
        in_specs=[hspec, hspec, hspec, hspec,
                  pl.BlockSpec((4, nc_in, 128, Ho), lambda i: (0, 0, 0, 0)),
                  pl.BlockSpec((1, Ho), lambda i: (0, 0))],
        out_specs=pl.BlockSpec((nc_out, BR, 128), lambda i: (0, i, 0)),
    )(*hs, W, b)


def kernel(x, edge_index, edge_attr, W1, b1, Wm, bm, W9, b9):
    key = (edge_index[1] // NREG) * 16384 + edge_index[0]
    perm = jnp.argsort(key)
    src = edge_index[0][perm]
    dst = edge_index[1][perm]
    ea = edge_attr[perm]
    pad = R * EB - E
    src_p = jnp.concatenate([src, jnp.zeros((pad,), jnp.int32)])
    dst_p = jnp.concatenate([dst, jnp.zeros((pad,), jnp.int32)]).reshape(R, EB)
    ea_p = jnp.concatenate([ea, jnp.zeros((pad,), jnp.float32)]).reshape(R, EB)
    src1 = src_p.reshape(R, EB)
    src4 = (src_p[None, :]
            + (jnp.arange(4, dtype=jnp.int32) * N)[:, None]).reshape(4 * R, EB)
    svals = jnp.searchsorted(
        dst // NREG, jnp.arange(33, dtype=jnp.int32)).astype(jnp.int32)
    bounds = jnp.zeros((256, 128), jnp.int32)
    rows8 = jnp.arange(32) * 8
    bounds = (bounds.at[rows8, 0].set(svals[:32])
              .at[rows8, 1].set(svals[1:]))

    hop1 = _make_hop(1)
    hop4 = _make_hop(4)
    norm_k = _make_norm()

    ones = jnp.ones((N, 128), jnp.float32)
    deg = hop1(ones, src1, dst_p, ea_p, bounds)
    dis = _dis_tc(deg)
    dis_col = lax.slice(dis, (0, 0), (N, 1)).reshape(N)
    nrm = norm_k(dis_col, src1, dst_p, ea_p)

    h0 = x
    h1 = hop1(h0, src1, dst_p, nrm, bounds)
    h2 = hop1(h1, src1, dst_p, nrm, bounds)
    h3 = hop1(h2, src1, dst_p, nrm, bounds)
    W1r = W1.reshape(4, 1, 128, H)
    h = _layer_tc([h0.reshape(1, N, 128), h1.reshape(1, N, 128),
                   h2.reshape(1, N, 128), h3.reshape(1, N, 128)],
                  W1r, b1.reshape(1, H), 1, 4, True)

    for i in range(7):
        f0 = h.reshape(4 * N, 128)
        f1 = hop4(f0, src4, dst_p, nrm, bounds)
        f2 = hop4(f1, src4, dst_p, nrm, bounds)
        f3 = hop4(f2, src4, dst_p, nrm, bounds)
        Wr = Wm[i].reshape(4, 4, 128, H)
        h = _layer_tc([h, f1.reshape(4, N, 128), f2.reshape(4, N, 128),
                       f3.reshape(4, N, 128)],
                      Wr, bm[i].reshape(1, H), 4, 4, True)

    f0 = h.reshape(4 * N, 128)
    f1 = hop4(f0, src4, dst_p, nrm, bounds)
    f2 = hop4(f1, src4, dst_p, nrm, bounds)
    f3 = hop4(f2, src4, dst_p, nrm, bounds)
    W9p = jnp.pad(W9, ((0, 0), (0, 0), (0, 128 - C)))
    b9p = jnp.pad(b9, (0, 128 - C))
    out = _layer_tc([h, f1.reshape(4, N, 128), f2.reshape(4, N, 128),
                     f3.reshape(4, N, 128)],
                    W9p.reshape(4, 4, 128, 128), b9p.reshape(1, 128),
                    4, 1, False)
    return out.reshape(N, 128)[:, :C]
